# Initial kernel scaffold; baseline (speedup 1.0000x reference)
import jax
import jax.numpy as jnp
from jax.experimental import pallas as pl


def kernel(V, R_s, R_r, dt, We1, be1, We2, be2, Wn1, bn1, Wn2, bn2, Wn3, bn3, Wo, bo):
    raise NotImplementedError("write your pallas kernel here")


# speedup vs baseline: 6.6733x; 6.6733x over previous
import functools

import jax
import jax.numpy as jnp
from jax import lax
from jax.experimental import pallas as pl
from jax.experimental.pallas import tpu as pltpu
from jax.experimental.pallas import tpu_sc as plsc

BOX = 6.0
HALF = BOX / 2.0

_NC, _NS = 2, 16
_NW = _NC * _NS


def _gather_rows(table, idx, chunk=2000):
    n, d = table.shape
    m = idx.shape[0]
    per_w = m // _NW
    assert per_w * _NW == m and per_w % chunk == 0 and chunk % 8 == 0
    nch = per_w // chunk
    mesh = plsc.VectorSubcoreMesh(
        core_axis_name="c", subcore_axis_name="s",
        num_cores=_NC, num_subcores=_NS)

    @functools.partial(
        pl.kernel,
        out_type=jax.ShapeDtypeStruct((m, d), jnp.float32),
        mesh=mesh,
        compiler_params=pltpu.CompilerParams(use_tc_tiling_on_sc=False),
        scratch_types=[
            pltpu.VMEM((chunk,), jnp.int32),
            pltpu.VMEM((chunk, d), jnp.float32),
            pltpu.SemaphoreType.DMA,
        ],
    )
    def gather_kernel(table_hbm, idx_hbm, out_hbm, idx_v, rows_v, sem):
        wid = lax.axis_index("s") * _NC + lax.axis_index("c")
        base = wid * per_w

        def body(i, carry):
            off = base + i * chunk
            pltpu.sync_copy(idx_hbm.at[pl.ds(off, chunk)], idx_v)
            pltpu.async_copy(table_hbm.at[idx_v], rows_v, sem).wait()
            pltpu.sync_copy(rows_v, out_hbm.at[pl.ds(off, chunk)])
            return carry

        lax.fori_loop(0, nch, body, 0)

    return gather_kernel(table, idx)


def _dot(a, b):
    return jnp.dot(a, b, preferred_element_type=jnp.float32)


def _tc_body(deg, gs_ref, gr_ref, v2_ref, w1s_ref, w1r_ref, w1d_ref, b1_ref,
             w2_ref, b2_ref, wn1a_ref, wn1v_ref, bn1_ref, wn2_ref, bn2_ref,
             wn3_ref, bn3_ref, wo_ref, bo_ref, out_ref):
    gs = gs_ref[...]
    gr = gr_ref[...]
    d = gs - gr
    d = jnp.where(d > HALF, d - BOX, d)
    d = jnp.where(d <= -HALF, d + BOX, d)
    h = _dot(gs, w1s_ref[...]) + _dot(gr, w1r_ref[...]) + _dot(d, w1d_ref[...])
    h = jnp.maximum(h + b1_ref[...], 0.0)
    en = jnp.maximum(_dot(h, w2_ref[...]) + b2_ref[...], 0.0)
    tn = out_ref.shape[0]
    agg = jnp.sum(en.reshape(tn, deg, en.shape[-1]), axis=1)
    v2 = v2_ref[...]
    z = jnp.maximum(_dot(agg, wn1a_ref[...]) + _dot(v2, wn1v_ref[...]) + bn1_ref[...], 0.0)
    z = jnp.maximum(_dot(z, wn2_ref[...]) + bn2_ref[...], 0.0)
    z = jnp.maximum(_dot(z, wn3_ref[...]) + bn3_ref[...], 0.0)
    newc = v2[:, 0:4] + _dot(z, wo_ref[...]) + bo_ref[...]
    cw = jnp.where(newc >= HALF, newc - BOX, newc)
    cw = jnp.where(cw < -HALF, cw + BOX, cw)
    lane = lax.broadcasted_iota(jnp.int32, newc.shape, 1)
    out_ref[...] = jnp.where(lane < 2, cw, newc)


def _tc_forward(g, v2, w1s, w1r, w1d, b1, w2, b2, wn1a, wn1v, bn1,
                wn2, bn2, wn3, bn3, wo, bo, tn, deg):
    n = v2.shape[0]
    te = tn * deg
    nb = n // tn
    assert nb * tn == n and g.shape[0] == 2 * n * deg

    def wspec(arr):
        return pl.BlockSpec(arr.shape, lambda i: tuple(0 for _ in arr.shape))

    grid_spec = pl.GridSpec(
        grid=(nb,),
        in_specs=[
            pl.BlockSpec((te, 8), lambda i: (i, 0)),
            pl.BlockSpec((te, 8), lambda i: (i + nb, 0)),
            pl.BlockSpec((tn, 8), lambda i: (i, 0)),
            wspec(w1s), wspec(w1r), wspec(w1d), wspec(b1),
            wspec(w2), wspec(b2), wspec(wn1a), wspec(wn1v), wspec(bn1),
            wspec(wn2), wspec(bn2), wspec(wn3), wspec(bn3),
            wspec(wo), wspec(bo),
        ],
        out_specs=pl.BlockSpec((tn, 4), lambda i: (i, 0)),
    )
    return pl.pallas_call(
        functools.partial(_tc_body, deg),
        grid_spec=grid_spec,
        out_shape=jax.ShapeDtypeStruct((n, 4), jnp.float32),
    )(g, g, v2, w1s, w1r, w1d, b1, w2, b2, wn1a, wn1v, bn1,
      wn2, bn2, wn3, bn3, wo, bo)


def kernel(V, R_s, R_r, dt, We1, be1, We2, be2, Wn1, bn1, Wn2, bn2,
           Wn3, bn3, Wo, bo):
    _, n, _ = V.shape
    e = R_s.shape[1]
    deg = e // n
    hd = We1.shape[1]
    nd = Wn1.shape[1]
    vf = V[0]
    dt0 = dt[0, 0]

    vno = jnp.concatenate([vf[:, 0:1], vf[:, 3:5]], axis=1)
    vtab = jnp.concatenate(
        [vno, jnp.zeros((n, 5), jnp.float32)], axis=1)
    v2 = jnp.concatenate(
        [vf[:, 3:7], vno, jnp.zeros((n, 1), jnp.float32)], axis=1)

    idx = jnp.concatenate([R_s[0], R_r[0]])
    g = _gather_rows(vtab, idx)

    z5 = jnp.zeros((5, hd), jnp.float32)
    w1s = jnp.concatenate([We1[0:3], z5], axis=0)
    w1r = jnp.concatenate([We1[3:6], z5], axis=0)
    w1d = jnp.concatenate(
        [jnp.zeros((1, hd), jnp.float32), We1[6:8],
         jnp.zeros((5, hd), jnp.float32)], axis=0)
    b1 = (be1 + dt0 * We1[8])[None, :]
    wn1a = Wn1[3:3 + hd]
    wn1v = jnp.concatenate(
        [jnp.zeros((4, nd), jnp.float32), Wn1[0:3],
         jnp.zeros((1, nd), jnp.float32)], axis=0)
    bn1d = (bn1 + dt0 * Wn1[3 + hd])[None, :]

    out = _tc_forward(g, v2, w1s, w1r, w1d, b1, We2, be2[None, :],
                      wn1a, wn1v, bn1d, Wn2, bn2[None, :], Wn3,
                      bn3[None, :], Wo, bo[None, :], tn=200, deg=deg)
    return out[None]


# speedup vs baseline: 6.6784x; 1.0008x over previous
import functools

import jax
import jax.numpy as jnp
from jax import lax
from jax.experimental import pallas as pl
from jax.experimental.pallas import tpu as pltpu
from jax.experimental.pallas import tpu_sc as plsc

BOX = 6.0
HALF = BOX / 2.0

_NC, _NS = 2, 16
_NW = _NC * _NS


def _gather_rows(table, idx, chunk=2000):
    n, d = table.shape
    m = idx.shape[0]
    per_w = m // _NW
    assert per_w * _NW == m and per_w % chunk == 0 and chunk % 8 == 0
    nch = per_w // chunk
    mesh = plsc.VectorSubcoreMesh(
        core_axis_name="c", subcore_axis_name="s",
        num_cores=_NC, num_subcores=_NS)

    @functools.partial(
        pl.kernel,
        out_type=jax.ShapeDtypeStruct((m, d), jnp.float32),
        mesh=mesh,
        compiler_params=pltpu.CompilerParams(use_tc_tiling_on_sc=False),
        scratch_types=[
            pltpu.VMEM((chunk,), jnp.int32),
            pltpu.VMEM((chunk, d), jnp.float32),
            pltpu.SemaphoreType.DMA,
        ],
    )
    def gather_kernel(table_hbm, idx_hbm, out_hbm, idx_v, rows_v, sem):
        wid = lax.axis_index("s") * _NC + lax.axis_index("c")
        base = wid * per_w

        def body(i, carry):
            off = base + i * chunk
            pltpu.sync_copy(idx_hbm.at[pl.ds(off, chunk)], idx_v)
            pltpu.async_copy(table_hbm.at[idx_v], rows_v, sem).wait()
            pltpu.sync_copy(rows_v, out_hbm.at[pl.ds(off, chunk)])
            return carry

        lax.fori_loop(0, nch, body, 0)

    return gather_kernel(table, idx)


def _dot(a, b):
    return jnp.dot(a, b, preferred_element_type=jnp.float32)


def _tc_body(deg, gs_ref, gr_ref, v2_ref, w1s_ref, w1r_ref, w1d_ref, b1_ref,
             w2_ref, b2_ref, wn1a_ref, wn1v_ref, bn1_ref, wn2_ref, bn2_ref,
             wn3_ref, bn3_ref, wo_ref, bo_ref, out_ref):
    gs = gs_ref[...]
    gr = gr_ref[...]
    d = gs - gr
    d = jnp.where(d > HALF, d - BOX, d)
    d = jnp.where(d <= -HALF, d + BOX, d)
    h = (_dot(gs.astype(jnp.bfloat16), w1s_ref[...])
         + _dot(gr.astype(jnp.bfloat16), w1r_ref[...])
         + _dot(d.astype(jnp.bfloat16), w1d_ref[...]))
    h = jnp.maximum(h + b1_ref[...], 0.0)
    en = jnp.maximum(_dot(h.astype(jnp.bfloat16), w2_ref[...]) + b2_ref[...], 0.0)
    tn = out_ref.shape[0]
    agg = jnp.sum(en.reshape(tn, deg, en.shape[-1]), axis=1)
    v2 = v2_ref[...]
    z = jnp.maximum(_dot(agg, wn1a_ref[...]) + _dot(v2, wn1v_ref[...]) + bn1_ref[...], 0.0)
    z = jnp.maximum(_dot(z, wn2_ref[...]) + bn2_ref[...], 0.0)
    z = jnp.maximum(_dot(z, wn3_ref[...]) + bn3_ref[...], 0.0)
    newc = v2[:, 0:4] + _dot(z, wo_ref[...]) + bo_ref[...]
    cw = jnp.where(newc >= HALF, newc - BOX, newc)
    cw = jnp.where(cw < -HALF, cw + BOX, cw)
    lane = lax.broadcasted_iota(jnp.int32, newc.shape, 1)
    out_ref[...] = jnp.where(lane < 2, cw, newc)


def _tc_forward(g, v2, w1s, w1r, w1d, b1, w2, b2, wn1a, wn1v, bn1,
                wn2, bn2, wn3, bn3, wo, bo, tn, deg):
    n = v2.shape[0]
    te = tn * deg
    nb = n // tn
    assert nb * tn == n and g.shape[0] == 2 * n * deg

    def wspec(arr):
        return pl.BlockSpec(arr.shape, lambda i: tuple(0 for _ in arr.shape))

    grid_spec = pl.GridSpec(
        grid=(nb,),
        in_specs=[
            pl.BlockSpec((te, 8), lambda i: (i, 0)),
            pl.BlockSpec((te, 8), lambda i: (i + nb, 0)),
            pl.BlockSpec((tn, 8), lambda i: (i, 0)),
            wspec(w1s), wspec(w1r), wspec(w1d), wspec(b1),
            wspec(w2), wspec(b2), wspec(wn1a), wspec(wn1v), wspec(bn1),
            wspec(wn2), wspec(bn2), wspec(wn3), wspec(bn3),
            wspec(wo), wspec(bo),
        ],
        out_specs=pl.BlockSpec((tn, 4), lambda i: (i, 0)),
    )
    return pl.pallas_call(
        functools.partial(_tc_body, deg),
        grid_spec=grid_spec,
        out_shape=jax.ShapeDtypeStruct((n, 4), jnp.float32),
    )(g, g, v2, w1s, w1r, w1d, b1, w2, b2, wn1a, wn1v, bn1,
      wn2, bn2, wn3, bn3, wo, bo)


def kernel(V, R_s, R_r, dt, We1, be1, We2, be2, Wn1, bn1, Wn2, bn2,
           Wn3, bn3, Wo, bo):
    _, n, _ = V.shape
    e = R_s.shape[1]
    deg = e // n
    hd = We1.shape[1]
    nd = Wn1.shape[1]
    vf = V[0]
    dt0 = dt[0, 0]

    vno = jnp.concatenate([vf[:, 0:1], vf[:, 3:5]], axis=1)
    vtab = jnp.concatenate(
        [vno, jnp.zeros((n, 5), jnp.float32)], axis=1)
    v2 = jnp.concatenate(
        [vf[:, 3:7], vno, jnp.zeros((n, 1), jnp.float32)], axis=1)

    idx = jnp.concatenate([R_s[0], R_r[0]])
    g = _gather_rows(vtab, idx)

    z5 = jnp.zeros((5, hd), jnp.float32)
    w1s = jnp.concatenate([We1[0:3], z5], axis=0).astype(jnp.bfloat16)
    w1r = jnp.concatenate([We1[3:6], z5], axis=0).astype(jnp.bfloat16)
    w1d = jnp.concatenate(
        [jnp.zeros((1, hd), jnp.float32), We1[6:8],
         jnp.zeros((5, hd), jnp.float32)], axis=0).astype(jnp.bfloat16)
    b1 = (be1 + dt0 * We1[8])[None, :]
    wn1a = Wn1[3:3 + hd]
    wn1v = jnp.concatenate(
        [jnp.zeros((4, nd), jnp.float32), Wn1[0:3],
         jnp.zeros((1, nd), jnp.float32)], axis=0)
    bn1d = (bn1 + dt0 * Wn1[3 + hd])[None, :]

    out = _tc_forward(g, v2, w1s, w1r, w1d, b1, We2.astype(jnp.bfloat16), be2[None, :],
                      wn1a, wn1v, bn1d, Wn2, bn2[None, :], Wn3,
                      bn3[None, :], Wo, bo[None, :], tn=200, deg=deg)
    return out[None]
